# baseline (speedup 1.0000x reference)
import functools

import jax
import jax.numpy as jnp
from jax import lax
from jax.experimental import pallas as pl
from jax.experimental.pallas import tpu as pltpu
from jax.experimental.pallas import tpu_sc as plsc

_NC = 2
_NS = 16
_NW = _NC * _NS
_LANES = 16
_RING = 4


@functools.cache
def _gather_pool_kernel(b: int, l: int, v: int, d: int):
    dh = d // 2
    chunk = 128
    rows_w = b // _NW
    per_w = rows_w * l
    n = b * l
    n_rg = per_w // (_RING * chunk)
    assert rows_w * _NW == b and n_rg * _RING * chunk == per_w
    stage_rows = 8
    n_stage = rows_w // stage_rows
    assert n_stage * stage_rows == rows_w
    nfull = l // _LANES
    tail = l - _LANES * nfull

    mesh = plsc.VectorSubcoreMesh(
        core_axis_name="c", subcore_axis_name="s",
        num_cores=_NC, num_subcores=_NS,
    )

    @functools.partial(
        pl.kernel,
        out_type=(
            jax.ShapeDtypeStruct((n, d), jnp.float32),
            jax.ShapeDtypeStruct((n, dh), jnp.float32),
        ),
        mesh=mesh,
        scratch_types=[
            pltpu.VMEM((stage_rows, l), jnp.int32),
            pltpu.VMEM((per_w,), jnp.int32),
            pltpu.VMEM((_RING, chunk, d), jnp.float32),
            pltpu.VMEM((2, chunk, dh), jnp.float32),
        ] + [pltpu.SemaphoreType.DMA] * (2 * _RING + 2),
        compiler_params=pltpu.CompilerParams(
            needs_layout_passes=False, use_tc_tiling_on_sc=True),
    )
    def gather_k(idx_hbm, t_hbm, x_hbm, p_hbm, idxb2, fl, xb, pb, *sems):
        sem_g, sem_wx, sem_wp = sems[:_RING], sems[_RING:2 * _RING], sems[2 * _RING:]
        wid = lax.axis_index("s") * _NC + lax.axis_index("c")
        base = wid * per_w
        row0 = wid * rows_w
        lane = lax.iota(jnp.int32, _LANES)

        def issue_gather(c, slot):
            iv = fl.at[pl.ds(c * chunk, chunk)]
            pltpu.async_copy(t_hbm.at[iv], xb.at[slot], sem_g[slot])

        def wait_gather(slot):
            iv = fl.at[pl.ds(0, chunk)]
            pltpu.make_async_copy(t_hbm.at[iv], xb.at[slot], sem_g[slot]).wait()

        def wait_wx(slot):
            pltpu.make_async_copy(
                xb.at[slot], x_hbm.at[pl.ds(0, chunk)], sem_wx[slot]).wait()

        def wait_wp(ps):
            pltpu.make_async_copy(
                pb.at[ps], p_hbm.at[pl.ds(0, chunk)], sem_wp[ps]).wait()

        def pool(slot, ps):
            def pool_row(r):
                rvec = jnp.broadcast_to(r, (_LANES,))
                for c2 in range(dh // _LANES):
                    ev = 32 * c2 + 2 * lane
                    e = plsc.load_gather(xb.at[slot], [rvec, ev])
                    o = plsc.load_gather(xb.at[slot], [rvec, ev + 1])
                    pb[ps, r, pl.ds(c2 * _LANES, _LANES)] = (
                        jnp.maximum(e, o))
            pl.loop(0, chunk)(pool_row)

        def stage_body(q):
            pltpu.sync_copy(
                idx_hbm.at[pl.ds(row0 + q * stage_rows, stage_rows), :], idxb2)

            def flat_row(r):
                fbase = q * stage_rows * l + r * l
                for kk in range(nfull):
                    fl[pl.ds(fbase + kk * _LANES, _LANES)] = (
                        idxb2[r, pl.ds(kk * _LANES, _LANES)])
                if tail:
                    fl[pl.ds(fbase + l - _LANES, _LANES)] = (
                        idxb2[r, pl.ds(l - _LANES, _LANES)])
            pl.loop(0, stage_rows)(flat_row)
        pl.loop(0, n_stage)(stage_body)

        for s in range(_RING - 1):
            issue_gather(s, s)

        def rg_body(rg):
            for s in range(_RING):
                c = rg * _RING + s
                wait_gather(s)
                s2 = (s + _RING - 1) % _RING
                if s == 0:
                    def pf0():
                        wait_wx(s2)
                        issue_gather(rg * _RING + _RING - 1, s2)
                    pl.when(rg > 0)(pf0)
                    pl.when(rg == 0)(
                        lambda: issue_gather(_RING - 1, s2))
                else:
                    def pf(s=s, s2=s2):
                        wait_wx(s2)
                        issue_gather((rg + 1) * _RING + s - 1, s2)
                    pl.when(rg < n_rg - 1)(pf)
                ps = s % 2
                if s < 2:
                    pl.when(rg > 0)(lambda ps=ps: wait_wp(ps))
                else:
                    wait_wp(ps)
                pool(s, ps)
                off = base + c * chunk
                pltpu.async_copy(xb.at[s], x_hbm.at[pl.ds(off, chunk)],
                                 sem_wx[s])
                pltpu.async_copy(pb.at[ps], p_hbm.at[pl.ds(off, chunk)],
                                 sem_wp[ps])

        pl.loop(0, n_rg)(rg_body)
        for s in range(_RING):
            wait_wx(s)
        for ps in range(2):
            wait_wp(ps)

    return gather_k


def kernel(inputs, table):
    b, l = inputs.shape
    v, d = table.shape
    x_flat, p_flat = _gather_pool_kernel(b, l, v, d)(inputs, table)
    return x_flat.reshape(b, l, d), p_flat.reshape(b, l, d // 2)


# speedup vs baseline: 1.0066x; 1.0066x over previous
import functools

import jax
import jax.numpy as jnp
from jax import lax
from jax.experimental import pallas as pl
from jax.experimental.pallas import tpu as pltpu
from jax.experimental.pallas import tpu_sc as plsc

_NC = 2
_NS = 16
_NW = _NC * _NS
_LANES = 16
_RING = 4


@functools.cache
def _gather_pool_kernel(n: int, v: int, d: int):
    dh = d // 2
    chunk = 128
    per_w = n // _NW
    n_chunk = per_w // chunk
    n_rg = n_chunk // _RING
    assert per_w * _NW == n and n_rg * _RING == n_chunk

    mesh = plsc.VectorSubcoreMesh(
        core_axis_name="c", subcore_axis_name="s",
        num_cores=_NC, num_subcores=_NS,
    )

    @functools.partial(
        pl.kernel,
        out_type=(
            jax.ShapeDtypeStruct((n, d), jnp.float32),
            jax.ShapeDtypeStruct((n, dh), jnp.float32),
        ),
        mesh=mesh,
        scratch_types=[
            pltpu.VMEM((n_chunk, chunk), jnp.int32),
            pltpu.VMEM((_RING, chunk, d), jnp.float32),
            pltpu.VMEM((2, chunk, dh), jnp.float32),
        ] + [pltpu.SemaphoreType.DMA] * (2 * _RING + 2),
        compiler_params=pltpu.CompilerParams(needs_layout_passes=False),
    )
    def gather_k(idx_hbm, t_hbm, x_hbm, p_hbm, idxb, xb, pb, *sems):
        sem_g, sem_wx, sem_wp = sems[:_RING], sems[_RING:2 * _RING], sems[2 * _RING:]
        wid = lax.axis_index("s") * _NC + lax.axis_index("c")
        base = wid * per_w
        lane = lax.iota(jnp.int32, _LANES)

        def issue_gather(c, slot):
            pltpu.async_copy(t_hbm.at[idxb.at[c, :]], xb.at[slot], sem_g[slot])

        def wait_gather(slot):
            pltpu.make_async_copy(
                t_hbm.at[idxb.at[0, :]], xb.at[slot], sem_g[slot]).wait()

        def wait_wx(slot):
            pltpu.make_async_copy(
                xb.at[slot], x_hbm.at[pl.ds(0, chunk)], sem_wx[slot]).wait()

        def wait_wp(ps):
            pltpu.make_async_copy(
                pb.at[ps], p_hbm.at[pl.ds(0, chunk)], sem_wp[ps]).wait()

        def pool(slot, ps):
            def pool_row(r):
                rvec = jnp.broadcast_to(r, (_LANES,))
                for c2 in range(dh // _LANES):
                    ev = 32 * c2 + 2 * lane
                    e = plsc.load_gather(xb.at[slot], [rvec, ev])
                    o = plsc.load_gather(xb.at[slot], [rvec, ev + 1])
                    pb[ps, r, pl.ds(c2 * _LANES, _LANES)] = (
                        jnp.maximum(e, o))
            pl.loop(0, chunk)(pool_row)

        pltpu.sync_copy(idx_hbm.at[pl.ds(wid * n_chunk, n_chunk), :], idxb)
        for s in range(_RING - 1):
            issue_gather(s, s)

        def rg_body(rg):
            for s in range(_RING):
                c = rg * _RING + s
                wait_gather(s)
                s2 = (s + _RING - 1) % _RING
                if s == 0:
                    def pf0():
                        wait_wx(s2)
                        issue_gather(rg * _RING + _RING - 1, s2)
                    pl.when(rg > 0)(pf0)
                    pl.when(rg == 0)(
                        lambda: issue_gather(_RING - 1, s2))
                else:
                    def pf(s=s, s2=s2):
                        wait_wx(s2)
                        issue_gather((rg + 1) * _RING + s - 1, s2)
                    pl.when(rg < n_rg - 1)(pf)
                ps = s % 2
                if s < 2:
                    pl.when(rg > 0)(lambda ps=ps: wait_wp(ps))
                else:
                    wait_wp(ps)
                pool(s, ps)
                off = base + c * chunk
                pltpu.async_copy(xb.at[s], x_hbm.at[pl.ds(off, chunk)],
                                 sem_wx[s])
                pltpu.async_copy(pb.at[ps], p_hbm.at[pl.ds(off, chunk)],
                                 sem_wp[ps])

        pl.loop(0, n_rg)(rg_body)
        for s in range(_RING):
            wait_wx(s)
        for ps in range(2):
            wait_wp(ps)

    return gather_k


def kernel(inputs, table):
    b, l = inputs.shape
    v, d = table.shape
    n = b * l
    idx128 = inputs.reshape(n // 128, 128)
    x_flat, p_flat = _gather_pool_kernel(n, v, d)(idx128, table)
    return x_flat.reshape(b, l, d), p_flat.reshape(b, l, d // 2)
